# baseline (speedup 1.0000x reference)
import jax
import jax.numpy as jnp
from jax import lax
from jax.experimental import pallas as pl
from jax.experimental.pallas import tpu as pltpu
from jax.experimental.pallas import tpu_sc as plsc

N = 320000
D = 128
NSEG = 1024
NC = 2
NS = 16
NW = NC * NS
SEG_PER_W = NSEG // NW
T = 384
NJ = D // 16
NEG_INF = float("-inf")


def _body(x_hbm, mem_hbm, out_sum_hbm, out_max_hbm,
          xbuf0, xbuf1, membuf0, membuf1, probe8, probe16,
          acc_sum, acc_max, sem0, sem1):
    wid = lax.axis_index("s") * NC + lax.axis_index("c")
    seg_base = wid * SEG_PER_W

    def ssearch2(v0, v1):
        def step(lo_b, hi_b, val, v):
            done = hi_b <= lo_b
            mid = jnp.minimum((lo_b + hi_b) // 2, N // 8 - 1)
            lo2 = jnp.where(done, lo_b, jnp.where(val < v, mid + 1, lo_b))
            hi2 = jnp.where(done, hi_b, jnp.where(val < v, hi_b, mid))
            return lo2, hi2

        def probe(lo_b, hi_b, dst, sm):
            mid = jnp.minimum((lo_b + hi_b) // 2, N // 8 - 1)
            return pltpu.async_copy(
                mem_hbm.at[pl.ds(pl.multiple_of(mid * 8, 8), 8)],
                dst.at[pl.ds(0, 8)], sm)

        def body(_, c):
            lo0, hi0, lo1, hi1 = c
            c0 = probe(lo0, hi0, probe8, sem0)
            c1 = probe(lo1, hi1, probe16, sem1)
            c0.wait()
            c1.wait()
            val0 = probe8[pl.ds(0, 16)][0]
            val1 = probe16[pl.ds(0, 16)][0]
            lo0, hi0 = step(lo0, hi0, val0, v0)
            lo1, hi1 = step(lo1, hi1, val1, v1)
            return (lo0, hi0, lo1, hi1)

        z = jnp.int32(0)
        nb = jnp.int32(N // 8)
        b0, _, b1, _ = lax.fori_loop(0, 16, body, (z, nb, z, nb))

        def refine(bstar, v, dst, sm):
            a = jnp.minimum(jnp.maximum(8 * (bstar - 1), 0), N - 16)
            return a, pltpu.async_copy(
                mem_hbm.at[pl.ds(pl.multiple_of(a, 8), 16)], dst, sm)

        a0, c0 = refine(b0, v0, probe8, sem0)
        a1, c1 = refine(b1, v1, probe16, sem1)
        c0.wait()
        c1.wait()

        def count(a, w, v):
            cnt = jnp.int32(0)
            for j in range(16):
                cnt = cnt + jnp.where(w[j] < v, jnp.int32(1), jnp.int32(0))
            return a + cnt

        return (count(a0, probe8[...], v0), count(a1, probe16[...], v1))

    lo, hi = ssearch2(jnp.int32(seg_base), jnp.int32(seg_base + SEG_PER_W))

    def init_body(s, _):
        for j in range(NJ):
            sl = pl.ds(j * 16, 16)
            acc_sum[s, sl] = jnp.zeros((16,), jnp.float32)
            acc_max[s, sl] = jnp.full((16,), NEG_INF, jnp.float32)
        return 0

    lax.fori_loop(0, SEG_PER_W, init_body, 0)

    nt = (hi - lo + (T - 1)) // T
    nt2 = ((nt + 1) // 2) * 2

    def a_of(t):
        start0 = lo + t * T
        a = pl.multiple_of(
            jnp.minimum((start0 // 8) * 8, N - T - 8), 8)
        return a, start0

    def issue(t, xb, mb, sm):
        @pl.when(t < nt2)
        def _():
            a, _ = a_of(t)
            pltpu.async_copy(x_hbm.at[pl.ds(a, T + 8)], xb, sm)
            pltpu.async_copy(mem_hbm.at[pl.ds(a, T + 8)],
                             mb.at[pl.ds(0, T + 8)], sm)

    def wait_compute(t, xb, mb, sm):
        a, start0 = a_of(t)
        pltpu.make_async_copy(x_hbm.at[pl.ds(a, T + 8)], xb, sm).wait()
        pltpu.make_async_copy(mem_hbm.at[pl.ds(a, T + 8)],
                              mb.at[pl.ds(0, T + 8)], sm).wait()
        i0 = start0 - a
        i1 = jnp.minimum(hi, start0 + T) - a

        def row_step(i, _):
            ls = mb[pl.ds(i, 16)][0] - seg_base
            for j in range(NJ):
                sl = pl.ds(j * 16, 16)
                xv = xb[i, sl]
                acc_sum[ls, sl] = acc_sum[ls, sl] + xv
                acc_max[ls, sl] = jnp.maximum(acc_max[ls, sl], xv)
            return 0

        GR = 32
        ng = jnp.maximum(i1 - i0, 0) // GR

        def g_body(g, _):
            gi = i0 + g * GR
            mva = mb[pl.ds(gi, 16)]
            mvb = mb[pl.ds(gi + GR - 16, 16)]
            uniform = mva[0] == mvb[15]

            @pl.when(uniform)
            def _():
                ls = mva[0] - seg_base
                s = [None] * NJ
                mx = [None] * NJ
                for j in range(NJ):
                    xv = xb[gi, pl.ds(j * 16, 16)]
                    s[j] = xv
                    mx[j] = xv
                for r in range(1, GR):
                    for j in range(NJ):
                        xv = xb[gi + r, pl.ds(j * 16, 16)]
                        s[j] = s[j] + xv
                        mx[j] = jnp.maximum(mx[j], xv)
                for j in range(NJ):
                    sl = pl.ds(j * 16, 16)
                    acc_sum[ls, sl] = acc_sum[ls, sl] + s[j]
                    acc_max[ls, sl] = jnp.maximum(acc_max[ls, sl], mx[j])

            @pl.when(jnp.logical_not(uniform))
            def _():
                lax.fori_loop(gi, gi + GR, row_step, 0)

            return 0

        lax.fori_loop(0, ng, g_body, 0)
        lax.fori_loop(i0 + ng * GR, i1, row_step, 0)

    issue(jnp.int32(0), xbuf0, membuf0, sem0)
    issue(jnp.int32(1), xbuf1, membuf1, sem1)

    def pair_body(p, _):
        t0 = 2 * p
        wait_compute(t0, xbuf0, membuf0, sem0)
        issue(t0 + 2, xbuf0, membuf0, sem0)
        wait_compute(t0 + 1, xbuf1, membuf1, sem1)
        issue(t0 + 3, xbuf1, membuf1, sem1)
        return 0

    lax.fori_loop(0, nt2 // 2, pair_body, 0)

    ob = pl.multiple_of(seg_base, 8)
    pltpu.sync_copy(acc_sum, out_sum_hbm.at[pl.ds(ob, SEG_PER_W)])
    pltpu.sync_copy(acc_max, out_max_hbm.at[pl.ds(ob, SEG_PER_W)])


@jax.jit
def _gather_pool(atom_features, membership_i32):
    mesh = plsc.VectorSubcoreMesh(
        core_axis_name="c", subcore_axis_name="s",
        num_cores=NC, num_subcores=NS)
    out_sum, out_max = pl.kernel(
        _body,
        out_type=(
            jax.ShapeDtypeStruct((NSEG, D), jnp.float32),
            jax.ShapeDtypeStruct((NSEG, D), jnp.float32),
        ),
        mesh=mesh,
        scratch_types=[
            pltpu.VMEM((T + 8, D), jnp.float32),
            pltpu.VMEM((T + 8, D), jnp.float32),
            pltpu.VMEM((T + 24,), jnp.int32),
            pltpu.VMEM((T + 24,), jnp.int32),
            pltpu.VMEM((16,), jnp.int32),
            pltpu.VMEM((16,), jnp.int32),
            pltpu.VMEM((SEG_PER_W, D), jnp.float32),
            pltpu.VMEM((SEG_PER_W, D), jnp.float32),
            pltpu.SemaphoreType.DMA,
            pltpu.SemaphoreType.DMA,
        ],
    )(atom_features, membership_i32)
    return jnp.concatenate([out_sum, out_max], axis=1)


def kernel(atom_features, unused_input1, membership):
    del unused_input1
    return _gather_pool(atom_features, membership.astype(jnp.int32))


# speedup vs baseline: 1.5052x; 1.5052x over previous
import jax
import jax.numpy as jnp
from jax import lax
from jax.experimental import pallas as pl
from jax.experimental.pallas import tpu as pltpu
from jax.experimental.pallas import tpu_sc as plsc

N = 320000
D = 128
NSEG = 1024
NC = 2
NS = 16
NW = NC * NS
SEG_PER_W = NSEG // NW
T = 384
NJ = D // 16
NEG_INF = float("-inf")


def _body(x_hbm, mem_hbm, out_sum_hbm, out_max_hbm,
          xbuf0, xbuf1, membuf0, membuf1, probe8, probe16,
          acc_sum, acc_max, sem0, sem1):
    wid = lax.axis_index("s") * NC + lax.axis_index("c")
    seg_base = wid * SEG_PER_W

    def ssearch2(v0, v1):
        def step(lo_b, hi_b, val, v):
            done = hi_b <= lo_b
            mid = jnp.minimum((lo_b + hi_b) // 2, N // 8 - 1)
            lo2 = jnp.where(done, lo_b, jnp.where(val < v, mid + 1, lo_b))
            hi2 = jnp.where(done, hi_b, jnp.where(val < v, hi_b, mid))
            return lo2, hi2

        def probe(lo_b, hi_b, dst, sm):
            mid = jnp.minimum((lo_b + hi_b) // 2, N // 8 - 1)
            return pltpu.async_copy(
                mem_hbm.at[pl.ds(pl.multiple_of(mid * 8, 8), 8)],
                dst.at[pl.ds(0, 8)], sm)

        def body(_, c):
            lo0, hi0, lo1, hi1 = c
            c0 = probe(lo0, hi0, probe8, sem0)
            c1 = probe(lo1, hi1, probe16, sem1)
            c0.wait()
            c1.wait()
            val0 = probe8[pl.ds(0, 16)][0]
            val1 = probe16[pl.ds(0, 16)][0]
            lo0, hi0 = step(lo0, hi0, val0, v0)
            lo1, hi1 = step(lo1, hi1, val1, v1)
            return (lo0, hi0, lo1, hi1)

        z = jnp.int32(0)
        nb = jnp.int32(N // 8)
        b0, _, b1, _ = lax.fori_loop(0, 16, body, (z, nb, z, nb))

        def refine(bstar, v, dst, sm):
            a = jnp.minimum(jnp.maximum(8 * (bstar - 1), 0), N - 16)
            return a, pltpu.async_copy(
                mem_hbm.at[pl.ds(pl.multiple_of(a, 8), 16)], dst, sm)

        a0, c0 = refine(b0, v0, probe8, sem0)
        a1, c1 = refine(b1, v1, probe16, sem1)
        c0.wait()
        c1.wait()

        def count(a, w, v):
            cnt = jnp.int32(0)
            for j in range(16):
                cnt = cnt + jnp.where(w[j] < v, jnp.int32(1), jnp.int32(0))
            return a + cnt

        return (count(a0, probe8[...], v0), count(a1, probe16[...], v1))

    lo, hi = ssearch2(jnp.int32(seg_base), jnp.int32(seg_base + SEG_PER_W))

    def init_body(s, _):
        for j in range(NJ):
            sl = pl.ds(j * 16, 16)
            acc_sum[s, sl] = jnp.zeros((16,), jnp.float32)
            acc_max[s, sl] = jnp.full((16,), NEG_INF, jnp.float32)
        return 0

    lax.fori_loop(0, SEG_PER_W, init_body, 0)

    nt = (hi - lo + (T - 1)) // T
    nt2 = ((nt + 1) // 2) * 2

    def a_of(t):
        start0 = lo + t * T
        a = pl.multiple_of(
            jnp.minimum((start0 // 8) * 8, N - T - 8), 8)
        return a, start0

    def issue(t, xb, mb, sm):
        @pl.when(t < nt2)
        def _():
            a, _ = a_of(t)
            pltpu.async_copy(x_hbm.at[pl.ds(a, T + 8)], xb, sm)
            pltpu.async_copy(mem_hbm.at[pl.ds(a, T + 8)],
                             mb.at[pl.ds(0, T + 8)], sm)

    def wait_compute(t, xb, mb, sm):
        a, start0 = a_of(t)
        pltpu.make_async_copy(x_hbm.at[pl.ds(a, T + 8)], xb, sm).wait()
        pltpu.make_async_copy(mem_hbm.at[pl.ds(a, T + 8)],
                              mb.at[pl.ds(0, T + 8)], sm).wait()
        i0 = start0 - a
        i1 = jnp.minimum(hi, start0 + T) - a

        def row_step(i, _):
            ls = mb[pl.ds(i, 16)][0] - seg_base
            for j in range(NJ):
                sl = pl.ds(j * 16, 16)
                xv = xb[i, sl]
                plsc.addupdate(acc_sum.at[ls, sl], xv)
                acc_max[ls, sl] = jnp.maximum(acc_max[ls, sl], xv)
            return 0

        ng = jnp.maximum(i1 - i0, 0) // 16

        def g_body(g, _):
            gi = i0 + g * 16
            mv = mb[pl.ds(gi, 16)]
            uniform = mv[0] == mv[15]

            @pl.when(uniform)
            def _():
                ls = mv[0] - seg_base
                s = [None] * NJ
                mx = [None] * NJ
                for j in range(NJ):
                    xv = xb[gi, pl.ds(j * 16, 16)]
                    s[j] = xv
                    mx[j] = xv
                for r in range(1, 16):
                    for j in range(NJ):
                        xv = xb[gi + r, pl.ds(j * 16, 16)]
                        s[j] = s[j] + xv
                        mx[j] = jnp.maximum(mx[j], xv)
                for j in range(NJ):
                    sl = pl.ds(j * 16, 16)
                    plsc.addupdate(acc_sum.at[ls, sl], s[j])
                    acc_max[ls, sl] = jnp.maximum(acc_max[ls, sl], mx[j])

            @pl.when(jnp.logical_not(uniform))
            def _():
                lax.fori_loop(gi, gi + 16, row_step, 0)

            return 0

        lax.fori_loop(0, ng, g_body, 0)
        lax.fori_loop(i0 + ng * 16, i1, row_step, 0)

    issue(jnp.int32(0), xbuf0, membuf0, sem0)
    issue(jnp.int32(1), xbuf1, membuf1, sem1)

    def pair_body(p, _):
        t0 = 2 * p
        wait_compute(t0, xbuf0, membuf0, sem0)
        issue(t0 + 2, xbuf0, membuf0, sem0)
        wait_compute(t0 + 1, xbuf1, membuf1, sem1)
        issue(t0 + 3, xbuf1, membuf1, sem1)
        return 0

    lax.fori_loop(0, nt2 // 2, pair_body, 0)

    ob = pl.multiple_of(seg_base, 8)
    pltpu.sync_copy(acc_sum, out_sum_hbm.at[pl.ds(ob, SEG_PER_W)])
    pltpu.sync_copy(acc_max, out_max_hbm.at[pl.ds(ob, SEG_PER_W)])


@jax.jit
def _gather_pool(atom_features, membership_i32):
    mesh = plsc.VectorSubcoreMesh(
        core_axis_name="c", subcore_axis_name="s",
        num_cores=NC, num_subcores=NS)
    out_sum, out_max = pl.kernel(
        _body,
        out_type=(
            jax.ShapeDtypeStruct((NSEG, D), jnp.float32),
            jax.ShapeDtypeStruct((NSEG, D), jnp.float32),
        ),
        mesh=mesh,
        scratch_types=[
            pltpu.VMEM((T + 8, D), jnp.float32),
            pltpu.VMEM((T + 8, D), jnp.float32),
            pltpu.VMEM((T + 24,), jnp.int32),
            pltpu.VMEM((T + 24,), jnp.int32),
            pltpu.VMEM((16,), jnp.int32),
            pltpu.VMEM((16,), jnp.int32),
            pltpu.VMEM((SEG_PER_W, D), jnp.float32),
            pltpu.VMEM((SEG_PER_W, D), jnp.float32),
            pltpu.SemaphoreType.DMA,
            pltpu.SemaphoreType.DMA,
        ],
    )(atom_features, membership_i32)
    return jnp.concatenate([out_sum, out_max], axis=1)


def kernel(atom_features, unused_input1, membership):
    del unused_input1
    return _gather_pool(atom_features, membership.astype(jnp.int32))
